# baseline (device time: 10299 ns/iter reference)
import jax
import jax.numpy as jnp
from jax import lax
from jax.experimental import pallas as pl
from jax.experimental.pallas import tpu as pltpu

_NBLK = 8


def kernel(x):
    m_per, n_per = x.shape
    bm = m_per // _NBLK
    rows = bm // 128

    def body(x_hbm, out_ref, buf, partial_ref, peer_ref,
             load_sems, send_sems, recv_sems):
        my_x = lax.axis_index("x")
        my_y = lax.axis_index("y")
        peer = (my_x, 1 - my_y)
        barrier_sem = pltpu.get_barrier_semaphore()

        def load(i):
            return pltpu.make_async_copy(
                x_hbm.at[pl.ds(i * bm, bm), :],
                buf.at[i % 2],
                load_sems.at[i % 2],
            )

        def block_rdma(k):
            return pltpu.make_async_remote_copy(
                src_ref=partial_ref.at[pl.ds(k * rows, rows)],
                dst_ref=peer_ref.at[pl.ds(k * rows, rows)],
                send_sem=send_sems.at[k],
                recv_sem=recv_sems.at[k],
                device_id=peer,
                device_id_type=pl.DeviceIdType.MESH,
            )

        pl.semaphore_signal(
            barrier_sem, inc=1, device_id=peer,
            device_id_type=pl.DeviceIdType.MESH,
        )

        load(0).start()
        for i in range(_NBLK):
            if i + 1 < _NBLK:
                load(i + 1).start()
            load(i).wait()
            t = buf[i % 2, :, 0:128].astype(jnp.float32)
            for g in range(1, n_per // 128):
                t = t + buf[i % 2, :, g * 128:(g + 1) * 128]
            prod = lax.dot_general(
                t.astype(jnp.bfloat16),
                jnp.ones((128, 128), jnp.bfloat16),
                (((1,), (0,)), ((), ())),
                preferred_element_type=jnp.float32,
            )
            partial_ref[pl.ds(i * rows, rows), :] = prod[:, 0].reshape(rows, 128)
            if i == 0:
                pl.semaphore_wait(barrier_sem, 1)
            block_rdma(i).start()

        for k in range(_NBLK):
            rdma = block_rdma(k)
            rdma.wait_send()
            rdma.wait_recv()
        out_ref[:, :] = partial_ref[:, :] + peer_ref[:, :]

    out = pl.pallas_call(
        body,
        out_shape=jax.ShapeDtypeStruct((m_per // 128, 128), jnp.float32),
        in_specs=[pl.BlockSpec(memory_space=pl.ANY)],
        out_specs=pl.BlockSpec(memory_space=pltpu.VMEM),
        scratch_shapes=[
            pltpu.VMEM((2, bm, n_per), x.dtype),
            pltpu.VMEM((m_per // 128, 128), jnp.float32),
            pltpu.VMEM((m_per // 128, 128), jnp.float32),
            pltpu.SemaphoreType.DMA((2,)),
            pltpu.SemaphoreType.DMA((_NBLK,)),
            pltpu.SemaphoreType.DMA((_NBLK,)),
        ],
        compiler_params=pltpu.CompilerParams(collective_id=0),
    )(x)
    return out.reshape(m_per, 1)


# device time: 8571 ns/iter; 1.2016x vs baseline; 1.2016x over previous
import jax
import jax.numpy as jnp
from jax import lax
from jax.experimental import pallas as pl
from jax.experimental.pallas import tpu as pltpu

_GRID = 4


def kernel(x):
    m_per, n_per = x.shape
    bm = m_per // _GRID
    rows = bm // 128

    def body(x_ref, out_ref, partial_ref, peer_ref, send_sems, recv_sems):
        i = pl.program_id(0)
        my_x = lax.axis_index("x")
        my_y = lax.axis_index("y")
        peer = (my_x, 1 - my_y)
        barrier_sem = pltpu.get_barrier_semaphore()

        def block_rdma(k):
            return pltpu.make_async_remote_copy(
                src_ref=partial_ref.at[pl.ds(k * rows, rows)],
                dst_ref=peer_ref.at[pl.ds(k * rows, rows)],
                send_sem=send_sems.at[k],
                recv_sem=recv_sems.at[k],
                device_id=peer,
                device_id_type=pl.DeviceIdType.MESH,
            )

        @pl.when(i == 0)
        def _():
            pl.semaphore_signal(
                barrier_sem, inc=1, device_id=peer,
                device_id_type=pl.DeviceIdType.MESH,
            )

        s = jnp.sum(x_ref[:, :].astype(jnp.float32), axis=1)
        partial_ref[pl.ds(i * rows, rows), :] = s.reshape(rows, 128)

        @pl.when(i == 0)
        def _():
            pl.semaphore_wait(barrier_sem, 1)

        block_rdma(i).start()

        @pl.when(i == _GRID - 1)
        def _():
            for k in range(_GRID):
                rdma = block_rdma(k)
                rdma.wait_send()
                rdma.wait_recv()
            out_ref[:, :] = partial_ref[:, :] + peer_ref[:, :]

    x = pltpu.with_memory_space_constraint(x, pltpu.MemorySpace.HBM)
    out = pl.pallas_call(
        body,
        grid=(_GRID,),
        out_shape=jax.ShapeDtypeStruct((m_per // 128, 128), jnp.float32),
        in_specs=[pl.BlockSpec((bm, n_per), lambda i: (i, 0))],
        out_specs=pl.BlockSpec((m_per // 128, 128), lambda i: (0, 0)),
        scratch_shapes=[
            pltpu.VMEM((m_per // 128, 128), jnp.float32),
            pltpu.VMEM((m_per // 128, 128), jnp.float32),
            pltpu.SemaphoreType.DMA((_GRID,)),
            pltpu.SemaphoreType.DMA((_GRID,)),
        ],
        compiler_params=pltpu.CompilerParams(collective_id=0),
    )(x)
    return out.reshape(m_per, 1)


# device time: 8397 ns/iter; 1.2265x vs baseline; 1.0207x over previous
import jax
import jax.numpy as jnp
from jax import lax
from jax.experimental import pallas as pl
from jax.experimental.pallas import tpu as pltpu

_NCHUNK = 4


def kernel(x):
    m_per, n_per = x.shape
    bm = m_per // _NCHUNK
    rows = bm // 128

    def body(x_ref, out_ref, partial_ref, peer_ref, send_sems, recv_sems):
        my_x = lax.axis_index("x")
        my_y = lax.axis_index("y")
        peer = (my_x, 1 - my_y)
        barrier_sem = pltpu.get_barrier_semaphore()

        def chunk_rdma(k):
            return pltpu.make_async_remote_copy(
                src_ref=partial_ref.at[pl.ds(k * rows, rows)],
                dst_ref=peer_ref.at[pl.ds(k * rows, rows)],
                send_sem=send_sems.at[k],
                recv_sem=recv_sems.at[k],
                device_id=peer,
                device_id_type=pl.DeviceIdType.MESH,
            )

        pl.semaphore_signal(
            barrier_sem, inc=1, device_id=peer,
            device_id_type=pl.DeviceIdType.MESH,
        )

        for k in range(_NCHUNK):
            s = jnp.sum(
                x_ref[pl.ds(k * bm, bm), :].astype(jnp.float32), axis=1
            )
            partial_ref[pl.ds(k * rows, rows), :] = s.reshape(rows, 128)
            if k == 0:
                pl.semaphore_wait(barrier_sem, 1)
            chunk_rdma(k).start()

        for k in range(_NCHUNK):
            rdma = chunk_rdma(k)
            rdma.wait_send()
            rdma.wait_recv()
        out_ref[:, :] = partial_ref[:, :] + peer_ref[:, :]

    out = pl.pallas_call(
        body,
        out_shape=jax.ShapeDtypeStruct((m_per // 128, 128), jnp.float32),
        in_specs=[pl.BlockSpec(memory_space=pltpu.VMEM)],
        out_specs=pl.BlockSpec(memory_space=pltpu.VMEM),
        scratch_shapes=[
            pltpu.VMEM((m_per // 128, 128), jnp.float32),
            pltpu.VMEM((m_per // 128, 128), jnp.float32),
            pltpu.SemaphoreType.DMA((_NCHUNK,)),
            pltpu.SemaphoreType.DMA((_NCHUNK,)),
        ],
        compiler_params=pltpu.CompilerParams(collective_id=0),
    )(x)
    return out.reshape(m_per, 1)
